# Initial kernel scaffold; baseline (speedup 1.0000x reference)
import jax
import jax.numpy as jnp
from jax.experimental import pallas as pl


def kernel(x_real, x_imag, attn_real, attn_imag, alpha):
    raise NotImplementedError("write your pallas kernel here")


# speedup vs baseline: 8.7263x; 8.7263x over previous
import functools

import jax
import jax.numpy as jnp
from jax import lax
from jax.experimental import pallas as pl
from jax.experimental.pallas import tpu as pltpu
from jax.experimental.pallas import tpu_sc as plsc

F = 64
D = 2 * F
B = 1024
TOPK = 32
BLK = 4096
CHUNK = 128
CPB = BLK // CHUNK
EPS = 1e-12
PAD_VAL = -2.0
NEG = -3.0


def _normalize_q(q):
    n = jnp.sqrt(jnp.sum(q * q, axis=1, keepdims=True))
    return q / jnp.maximum(n, EPS)


def _block_sims(q_ref, dbt_ref, n_real):
    q_n = _normalize_q(q_ref[...])
    blk = dbt_ref[...]
    col_norm2 = jnp.sum(blk * blk, axis=0, keepdims=True)
    blk_n = blk / jnp.maximum(jnp.sqrt(col_norm2), EPS)
    s = jax.lax.dot_general(
        q_n.astype(jnp.bfloat16), blk_n.astype(jnp.bfloat16),
        (((1,), (0,)), ((), ())),
        preferred_element_type=jnp.float32,
    )
    b = pl.program_id(0)
    col = b * BLK + jax.lax.broadcasted_iota(jnp.int32, (1, BLK), 1)
    return jnp.where(col < n_real, s, PAD_VAL)


def _sims_kernel(q_ref, dbt_ref, sims_ref, fids_ref, p_acc, *,
                 n_real, n_blocks):
    s = _block_sims(q_ref, dbt_ref, n_real)
    b = pl.program_id(0)
    win = pl.multiple_of((b // 4) * 128, 128)
    sub = (b % 4) * CPB
    lane = jax.lax.broadcasted_iota(jnp.int32, (B, 128), 1)
    base = jnp.where(b % 4 == 0,
                     jnp.full((B, 128), NEG, jnp.float32),
                     p_acc[:, pl.ds(win, 128)])
    for r in range(CPB):
        chunk = s[:, r * CHUNK:(r + 1) * CHUNK]
        sims_ref[pl.ds(r * B, B), :] = chunk
        m = jnp.max(chunk, axis=1, keepdims=True)
        base = jnp.where(lane == sub + r, m, base)
    p_acc[:, pl.ds(win, 128)] = base

    @pl.when(b == n_blocks - 1)
    def _extract():
        p = p_acc[...]
        lane = jax.lax.broadcasted_iota(jnp.int32, p.shape, 1)
        kiota = jax.lax.broadcasted_iota(jnp.int32, (B, TOPK), 1)
        qiota = jax.lax.broadcasted_iota(jnp.int32, (B, TOPK), 0)
        acc_ids = jnp.zeros((B, TOPK), jnp.int32)
        for k in range(TOPK):
            m = jnp.max(p, axis=1, keepdims=True)
            gid = jnp.min(jnp.where(p == m, lane, jnp.int32(2 ** 30)),
                          axis=1, keepdims=True)
            acc_ids = jnp.where(kiota == k, gid, acc_ids)
            p = jnp.where(lane == gid, NEG, p)
        fids_ref[...] = acc_ids * B + qiota


def _agg_kernel(q_ref, dbt_ref, db_ref, cand_ref, alpha_ref,
                out_re_ref, out_im_ref,
                v32_scr, acc_rho, acc_sin, acc_cos, acc_cnt, *,
                n_real, n_blocks):
    b = pl.program_id(0)

    @pl.when(b == 0)
    def _init():
        def bis(_, carry):
            lo, hi = carry
            mid = 0.5 * (lo + hi)
            cnt = jnp.sum((cand_ref[...] >= mid).astype(jnp.float32),
                          axis=1, keepdims=True)
            ge = cnt >= TOPK
            return jnp.where(ge, mid, lo), jnp.where(ge, hi, mid)

        lo, hi = lax.fori_loop(
            0, 48, bis,
            (jnp.full((B, 1), -1.01, jnp.float32),
             jnp.full((B, 1), 1.01, jnp.float32)))
        c_ = cand_ref[...]
        v32_scr[...] = jnp.min(jnp.where(c_ >= lo, c_, 2.0),
                               axis=1, keepdims=True)
        acc_rho[...] = jnp.zeros_like(acc_rho)
        acc_sin[...] = jnp.zeros_like(acc_sin)
        acc_cos[...] = jnp.zeros_like(acc_cos)
        acc_cnt[...] = jnp.zeros_like(acc_cnt)

    s = _block_sims(q_ref, dbt_ref, n_real)
    mask = (s >= v32_scr[...]).astype(jnp.float32)

    blk = db_ref[...]
    re = blk[:, :F]
    im = blk[:, F:]
    r0 = jnp.sqrt(re * re + im * im)
    rho = r0 + 1e-07
    rinv = 1.0 / jnp.maximum(r0, 1e-30)
    sb = im * rinv
    cb = re * rinv

    dot = lambda a, x: jax.lax.dot_general(
        a, x, (((1,), (0,)), ((), ())), preferred_element_type=jnp.float32)
    acc_rho[...] += dot(mask, rho)
    acc_sin[...] += dot(mask, sb)
    acc_cos[...] += dot(mask, cb)
    acc_cnt[...] += jnp.sum(mask, axis=1, keepdims=True)

    @pl.when(b == n_blocks - 1)
    def _epilogue():
        cnt = jnp.maximum(acc_cnt[...], 1.0)
        mean_rho = acc_rho[...] / cnt
        ssum = acc_sin[...]
        csum = acc_cos[...]
        h = jnp.sqrt(ssum * ssum + csum * csum)
        good = h > 0.0
        hs = jnp.where(good, h, 1.0)
        cphi = jnp.where(good, csum / hs, 1.0)
        sphi = jnp.where(good, ssum / hs, 0.0)
        agg_re = mean_rho * cphi
        agg_im = mean_rho * sphi
        a = jnp.clip(alpha_ref[0, 0], 0.0, 1.0)
        q = q_ref[...]
        out_re_ref[...] = (1.0 - a) * q[:, :F] + a * agg_re
        out_im_ref[...] = (1.0 - a) * q[:, F:] + a * agg_im


def _make_sc_gather():
    mesh = plsc.VectorSubcoreMesh(core_axis_name="c", subcore_axis_name="s")
    q_per_worker = B // 32

    @functools.partial(
        pl.kernel,
        mesh=mesh,
        out_type=jax.ShapeDtypeStruct((B, TOPK, CHUNK), jnp.float32),
        scratch_types=[
            pltpu.VMEM((TOPK,), jnp.int32),
            pltpu.VMEM((TOPK, CHUNK), jnp.float32),
            pltpu.SemaphoreType.DMA,
            pltpu.SemaphoreType.DMA,
        ],
    )
    def sc_gather(fids_hbm, sims_hbm, out_hbm, idx_v, rows_v, sem_i, sem_g):
        wid = lax.axis_index("s") * 2 + lax.axis_index("c")

        def body(qloc, carry):
            qq = wid * q_per_worker + qloc
            pltpu.make_async_copy(fids_hbm.at[qq], idx_v, sem_i).start()
            pltpu.make_async_copy(fids_hbm.at[qq], idx_v, sem_i).wait()
            cpy = pltpu.make_async_copy(sims_hbm.at[idx_v], rows_v, sem_g)
            cpy.start()
            cpy.wait()
            out_cpy = pltpu.make_async_copy(rows_v, out_hbm.at[qq], sem_i)
            out_cpy.start()
            out_cpy.wait()
            return carry

        lax.fori_loop(0, q_per_worker, body, 0)

    return sc_gather


def kernel(x_real, x_imag, attn_real, attn_imag, alpha):
    n_real = B + attn_real.shape[0]
    n_blocks = pl.cdiv(n_real, BLK)
    n_pad = n_blocks * BLK
    n_rows = (n_pad // CHUNK) * B

    q = jnp.concatenate([x_real, x_imag], axis=1)
    db_re = jnp.concatenate([x_real, attn_real], axis=0)
    db_im = jnp.concatenate([x_imag, attn_imag], axis=0)
    db = jnp.concatenate([db_re, db_im], axis=1)
    db = jnp.pad(db, ((0, n_pad - n_real), (0, 0)))
    dbt = db.T
    alpha_arr = jnp.reshape(alpha, (1, 1)).astype(jnp.float32)

    q_spec = pl.BlockSpec((B, D), lambda b: (0, 0))
    dbt_spec = pl.BlockSpec((D, BLK), lambda b: (0, b))

    sims, fids = pl.pallas_call(
        functools.partial(_sims_kernel, n_real=n_real, n_blocks=n_blocks),
        grid=(n_blocks,),
        in_specs=[q_spec, dbt_spec],
        out_specs=[
            pl.BlockSpec((CPB * B, CHUNK), lambda b: (b, 0)),
            pl.BlockSpec((B, TOPK), lambda b: (0, 0)),
        ],
        out_shape=[
            jax.ShapeDtypeStruct((n_rows, CHUNK), jnp.float32),
            jax.ShapeDtypeStruct((B, TOPK), jnp.int32),
        ],
        scratch_shapes=[pltpu.VMEM((B, pl.cdiv(n_blocks, 4) * 128),
                                   jnp.float32)],
    )(q, dbt)

    cand = _make_sc_gather()(fids, sims)
    cand = jnp.reshape(cand, (B, TOPK * CHUNK))

    out_re, out_im = pl.pallas_call(
        functools.partial(_agg_kernel, n_real=n_real, n_blocks=n_blocks),
        grid=(n_blocks,),
        in_specs=[
            q_spec,
            dbt_spec,
            pl.BlockSpec((BLK, D), lambda b: (b, 0)),
            pl.BlockSpec((B, TOPK * CHUNK), lambda b: (0, 0)),
            pl.BlockSpec(memory_space=pltpu.SMEM),
        ],
        out_specs=[
            pl.BlockSpec((B, F), lambda b: (0, 0)),
            pl.BlockSpec((B, F), lambda b: (0, 0)),
        ],
        out_shape=[
            jax.ShapeDtypeStruct((B, F), jnp.float32),
            jax.ShapeDtypeStruct((B, F), jnp.float32),
        ],
        scratch_shapes=[
            pltpu.VMEM((B, 1), jnp.float32),
            pltpu.VMEM((B, F), jnp.float32),
            pltpu.VMEM((B, F), jnp.float32),
            pltpu.VMEM((B, F), jnp.float32),
            pltpu.VMEM((B, 1), jnp.float32),
        ],
    )(q, dbt, db, cand, alpha_arr)

    return jnp.concatenate([out_re, out_im], axis=1)
